# Initial kernel scaffold; baseline (speedup 1.0000x reference)
import jax
import jax.numpy as jnp
from jax.experimental import pallas as pl


def kernel(inputs, weight):
    raise NotImplementedError("write your pallas kernel here")


# speedup vs baseline: 8.0267x; 8.0267x over previous
import functools

import jax
import jax.numpy as jnp
from jax import lax
from jax.experimental import pallas as pl
from jax.experimental.pallas import tpu as pltpu
from jax.experimental.pallas import tpu_sc as plsc

B, C, T = 16, 256, 1024
N = B * T
K = 8192
D = 256
TB = 256
NBLK = N // TB
COMMIT = 0.25


def _argmin_body(x_ref, w_ref, c_ref, idx_ref, enc_ref, perp_ref, cnt_ref):
    i = pl.program_id(0)
    x = x_ref[...]
    s = jnp.sum(x * x, axis=1, keepdims=True)
    m = lax.dot_general(x, w_ref[...], (((1,), (1,)), ((), ())),
                        preferred_element_type=jnp.float32)
    d = (s + c_ref[...]) - 2.0 * m
    dmin = jnp.min(d, axis=1, keepdims=True)
    col = lax.broadcasted_iota(jnp.int32, (TB, K), 1)
    idx = jnp.min(jnp.where(d == dmin, col, K), axis=1, keepdims=True)
    idx_ref[...] = idx
    enc = jnp.where(col == idx, 1.0, 0.0).astype(jnp.float32)
    enc_ref[...] = enc
    blk_cnt = jnp.sum(enc, axis=0, keepdims=True)

    @pl.when(i == 0)
    def _():
        cnt_ref[...] = blk_cnt

    @pl.when(i > 0)
    def _():
        cnt_ref[...] += blk_cnt

    @pl.when(i == NBLK - 1)
    def _():
        avg = cnt_ref[...] * (1.0 / N)
        ent = -jnp.sum(avg * jnp.log(avg + 1e-10), axis=(0, 1), keepdims=True)
        perp_ref[...] = jnp.exp(ent)


def _run_argmin(x2d, weight, csq):
    return pl.pallas_call(
        _argmin_body,
        grid=(NBLK,),
        in_specs=[
            pl.BlockSpec((TB, D), lambda i: (i, 0)),
            pl.BlockSpec((K, D), lambda i: (0, 0)),
            pl.BlockSpec((1, K), lambda i: (0, 0)),
        ],
        out_specs=[
            pl.BlockSpec((TB, 1), lambda i: (i, 0)),
            pl.BlockSpec((TB, K), lambda i: (i, 0)),
            pl.BlockSpec((1, 1), lambda i: (0, 0)),
        ],
        out_shape=[
            jax.ShapeDtypeStruct((N, 1), jnp.int32),
            jax.ShapeDtypeStruct((N, K), jnp.float32),
            jax.ShapeDtypeStruct((1, 1), jnp.float32),
        ],
        scratch_shapes=[pltpu.VMEM((1, K), jnp.float32)],
    )(x2d, weight, csq)


_SC_CHUNK = 128


def _sc_gather_body(idx_hbm, w_hbm, out_hbm, idx_v, rows_v, sem):
    wid = lax.axis_index("s") * 2 + lax.axis_index("c")
    per_w = N // 32
    for chunk in range(per_w // _SC_CHUNK):
        base = wid * per_w + chunk * _SC_CHUNK
        pltpu.sync_copy(idx_hbm.at[pl.ds(base, _SC_CHUNK)], idx_v)
        pltpu.async_copy(w_hbm.at[idx_v], rows_v, sem).wait()
        pltpu.sync_copy(rows_v, out_hbm.at[pl.ds(base, _SC_CHUNK)])


def _run_gather(idx_flat, weight):
    mesh = plsc.VectorSubcoreMesh(core_axis_name="c", subcore_axis_name="s")
    fn = functools.partial(
        pl.kernel,
        mesh=mesh,
        out_type=jax.ShapeDtypeStruct((N, D), jnp.float32),
        scratch_types=[
            pltpu.VMEM((_SC_CHUNK,), jnp.int32),
            pltpu.VMEM((_SC_CHUNK, D), jnp.float32),
            pltpu.SemaphoreType.DMA,
        ],
    )(_sc_gather_body)
    return fn(idx_flat, weight)


def _out_body(x_ref, q_ref, out_ref, loss_ref, acc_ref):
    i = pl.program_id(0)
    xb = x_ref[0]
    qb = q_ref[...]
    qt = qb.T
    diff = qt - xb
    out_ref[0] = xb + diff
    sq = jnp.sum(diff * diff, axis=(0, 1), keepdims=True)

    @pl.when(i == 0)
    def _():
        acc_ref[...] = sq

    @pl.when(i > 0)
    def _():
        acc_ref[...] += sq

    @pl.when(i == NBLK - 1)
    def _():
        mse = acc_ref[...] * (1.0 / (N * D))
        loss_ref[...] = (1.0 + COMMIT) * mse


def _run_out(inputs, q):
    tpb = T // TB
    return pl.pallas_call(
        _out_body,
        grid=(NBLK,),
        in_specs=[
            pl.BlockSpec((1, C, TB), lambda i: (i // tpb, 0, i % tpb)),
            pl.BlockSpec((TB, D), lambda i: (i, 0)),
        ],
        out_specs=[
            pl.BlockSpec((1, C, TB), lambda i: (i // tpb, 0, i % tpb)),
            pl.BlockSpec((1, 1), lambda i: (0, 0)),
        ],
        out_shape=[
            jax.ShapeDtypeStruct((B, C, T), jnp.float32),
            jax.ShapeDtypeStruct((1, 1), jnp.float32),
        ],
        scratch_shapes=[pltpu.VMEM((1, 1), jnp.float32)],
    )(inputs, q)


def kernel(inputs, weight):
    x2d = jnp.transpose(inputs, (0, 2, 1)).reshape(N, D)
    csq = jnp.sum(weight ** 2, axis=1).reshape(1, K)
    idx, encodings, perp = _run_argmin(x2d, weight, csq)
    q = _run_gather(idx.reshape(N), weight)
    quantized_out, loss = _run_out(inputs, q)
    return (loss[0, 0], quantized_out, perp[0, 0], weight,
            idx.reshape(B, T), encodings)
